# baseline (speedup 1.0000x reference)
import jax
import jax.numpy as jnp
from jax import lax
from jax.experimental import pallas as pl
from jax.experimental.pallas import tpu as pltpu
from jax.experimental.pallas import tpu_sc as plsc

B = 4096
L = 200
H = 64
TYPE_VOCAB = 4
EPS = 1e-12

NW = 32
BATCH_PER_W = B // NW
GB = 8
CROWS = GB * L
NCHUNK = BATCH_PER_W // GB
BLK = 8


def _rsqrt(x):
    i = plsc.bitcast(x, jnp.int32)
    i = jnp.int32(0x5F3759DF) - (i >> 1)
    y = plsc.bitcast(i, jnp.float32)
    for _ in range(3):
        y = y * (1.5 - 0.5 * x * y * y)
    return y


def _body(seq_hbm, tid_hbm, table_hbm, type_hbm, gamma_hbm, beta_hbm,
          out_hbm, idx_v, tid_v, rows_v, type_v, gamma_v, beta_v, sems,
          wsem, isem):
    wid = lax.axis_index("s") * 2 + lax.axis_index("c")
    base = wid * BATCH_PER_W

    pltpu.sync_copy(type_hbm, type_v)
    pltpu.sync_copy(gamma_hbm, gamma_v)
    pltpu.sync_copy(beta_hbm, beta_v)

    lane = lax.iota(jnp.int32, 16)
    inv_h = jnp.float32(1.0 / H)

    b_first = pl.multiple_of(base, 8)
    pltpu.async_copy(seq_hbm.at[pl.ds(b_first, GB)], idx_v.at[0], isem)
    pltpu.async_copy(tid_hbm.at[pl.ds(b_first, GB)], tid_v.at[0], isem)

    def chunk_body(g, carry):
        b0 = pl.multiple_of(base + g * GB, 8)
        p = g & 1
        pltpu.make_async_copy(seq_hbm.at[pl.ds(b0, GB)], idx_v.at[p],
                              isem).wait()
        pltpu.make_async_copy(tid_hbm.at[pl.ds(b0, GB)], tid_v.at[p],
                              isem).wait()
        gn = jnp.minimum(g + 1, NCHUNK - 1)
        bn = pl.multiple_of(base + gn * GB, 8)
        pltpu.async_copy(seq_hbm.at[pl.ds(bn, GB)], idx_v.at[1 - p], isem)
        pltpu.async_copy(tid_hbm.at[pl.ds(bn, GB)], tid_v.at[1 - p], isem)
        @pl.when(g > 0)
        def _():
            for j in range(GB):
                pltpu.make_async_copy(
                    out_hbm.at[b0 + j], rows_v.at[pl.ds(j * L, L)],
                    wsem).wait()
        quarters = []
        for q in range(GB // 2):
            cq = []
            for j in (2 * q, 2 * q + 1):
                for off, sz in ((0, 104), (104, 96)):
                    cq.append(pltpu.async_copy(
                        table_hbm.at[idx_v.at[p, j, pl.ds(off, sz)]],
                        rows_v.at[pl.ds(j * L + off, sz)], sems.at[q]))
            quarters.append(cq)

        def group_body(gi, c2):
            j_ids, l_ids = c2
            row_ids = j_ids * L + l_ids
            tbase = plsc.load_gather(tid_v.at[p], [j_ids, l_ids]) * H
            acc = [jnp.zeros((16,), jnp.float32) for _ in range(2)]
            accq = [jnp.zeros((16,), jnp.float32) for _ in range(2)]
            for blk in range(H // BLK):
                vals, hhs = [], []
                for i in range(BLK):
                    d = blk * BLK + i
                    hh = (l_ids + d) & (H - 1)
                    v = plsc.load_gather(rows_v, [row_ids, hh])
                    t = plsc.load_gather(type_v, [tbase + hh])
                    val = v + t
                    acc[i & 1] = acc[i & 1] + val
                    accq[i & 1] = accq[i & 1] + val * val
                    vals.append(val)
                    hhs.append(hh)
                for i in range(BLK):
                    plsc.store_scatter(rows_v, [row_ids, hhs[i]], vals[i])
            mean = (acc[0] + acc[1]) * inv_h
            ssum = accq[0] + accq[1]
            var = ssum * inv_h - mean * mean
            rstd = _rsqrt(var + EPS)
            nmr = mean * rstd
            for blk in range(H // BLK):
                outs, hhs = [], []
                for i in range(BLK):
                    d = blk * BLK + i
                    hh = (l_ids + d) & (H - 1)
                    v = plsc.load_gather(rows_v, [row_ids, hh])
                    gm = plsc.load_gather(gamma_v, [hh])
                    bt = plsc.load_gather(beta_v, [hh])
                    outs.append((v * rstd - nmr) * gm + bt)
                    hhs.append(hh)
                for i in range(BLK):
                    plsc.store_scatter(rows_v, [row_ids, hhs[i]], outs[i])
            l_nxt = l_ids + 16
            wrap = l_nxt >= L
            l_nxt = jnp.where(wrap, l_nxt - L, l_nxt)
            j_nxt = jnp.where(wrap, j_ids + 1, j_ids)
            return (j_nxt, l_nxt)

        for q in range(GB // 2):
            for cp in quarters[q]:
                cp.wait()
            lax.fori_loop(0, 2 * L // 16, group_body,
                          (jnp.full((16,), 2 * q, jnp.int32), lane))
            for j in (2 * q, 2 * q + 1):
                pltpu.async_copy(
                    rows_v.at[pl.ds(j * L, L)], out_hbm.at[b0 + j], wsem)
        return carry

    lax.fori_loop(0, NCHUNK, chunk_body, 0)
    b_last = pl.multiple_of(base + (NCHUNK - 1) * GB, 8)
    for j in range(GB):
        pltpu.make_async_copy(
            out_hbm.at[b_last + j], rows_v.at[pl.ds(j * L, L)], wsem).wait()
    pltpu.make_async_copy(seq_hbm.at[pl.ds(b_last, GB)],
                          idx_v.at[0], isem).wait()
    pltpu.make_async_copy(tid_hbm.at[pl.ds(b_last, GB)],
                          tid_v.at[0], isem).wait()


@jax.jit
def _run(seq, type_ids, seq_table, type_flat, gamma, beta):
    mesh = plsc.VectorSubcoreMesh(core_axis_name="c", subcore_axis_name="s")
    k = pl.kernel(
        _body,
        out_type=jax.ShapeDtypeStruct((B, L, H), jnp.float32),
        mesh=mesh,
        scratch_types=[
            pltpu.VMEM((2, GB, L), jnp.int32),
            pltpu.VMEM((2, GB, L), jnp.int32),
            pltpu.VMEM((GB * L, H), jnp.float32),
            pltpu.VMEM((TYPE_VOCAB * H,), jnp.float32),
            pltpu.VMEM((H,), jnp.float32),
            pltpu.VMEM((H,), jnp.float32),
            pltpu.SemaphoreType.DMA((GB // 2,)),
            pltpu.SemaphoreType.DMA,
            pltpu.SemaphoreType.DMA,
        ],
        compiler_params=pltpu.CompilerParams(
            use_tc_tiling_on_sc=False,
            needs_layout_passes=False,
        ),
    )
    return k(seq, type_ids, seq_table, type_flat, gamma, beta)


def kernel(seq, type_ids, seq_table, type_table, gamma, beta):
    seq_i = seq.astype(jnp.int32)
    tid_i = type_ids.astype(jnp.int32)
    type_flat = type_table.reshape(TYPE_VOCAB * H)
    return _run(seq_i, tid_i, seq_table, type_flat, gamma, beta)
